# initial kernel scaffold (unmeasured)
import jax
import jax.numpy as jnp
from jax import lax
from jax.experimental import pallas as pl
from jax.experimental.pallas import tpu as pltpu

N_DEV = 4


def kernel(x, w_mat, scale_x, scale_w):
    m_per, k = x.shape
    n_total = w_mat.shape[1]
    n_per = n_total // N_DEV
    my = lax.axis_index("i")

    x8 = x.astype(jnp.float8_e5m2)
    w8 = lax.dynamic_slice(w_mat, (0, my * n_per), (k, n_per)).astype(
        jnp.float8_e5m2
    )
    sx = scale_x.astype(jnp.float32)
    sw = scale_w.astype(jnp.float32)

    def body(x_ref, w_ref, sx_ref, sw_ref, out_ref, comm_ref, send_sems, recv_sems):
        my_pos = lax.axis_index("i")
        left = lax.rem(my_pos + N_DEV - 1, N_DEV)
        right = lax.rem(my_pos + 1, N_DEV)

        barrier_sem = pltpu.get_barrier_semaphore()
        for nbr in (left, right):
            pl.semaphore_signal(
                barrier_sem, inc=1,
                device_id=(nbr,), device_id_type=pl.DeviceIdType.MESH,
            )
        pl.semaphore_wait(barrier_sem, 2)

        scale = sx_ref[0] * sw_ref[0]

        def block(x_chunk, origin):
            acc = jnp.dot(x_chunk, w_ref[...], preferred_element_type=jnp.float32)
            y = acc * scale
            z = y / (1.0 + jnp.exp(-jnp.clip(y, -60.0, 60.0)))
            out_ref[pl.ds(origin * m_per, m_per), :] = z

        hop0 = pltpu.make_async_remote_copy(
            src_ref=x_ref, dst_ref=comm_ref.at[0],
            send_sem=send_sems.at[0], recv_sem=recv_sems.at[0],
            device_id=(right,), device_id_type=pl.DeviceIdType.MESH,
        )
        hop0.start()
        block(x_ref[...], my_pos)
        hop0.wait()

        for h in (1, 2):
            rdma = pltpu.make_async_remote_copy(
                src_ref=comm_ref.at[h - 1], dst_ref=comm_ref.at[h],
                send_sem=send_sems.at[h], recv_sem=recv_sems.at[h],
                device_id=(right,), device_id_type=pl.DeviceIdType.MESH,
            )
            rdma.start()
            block(comm_ref[h - 1], lax.rem(my_pos + N_DEV - h, N_DEV))
            rdma.wait()

        block(comm_ref[2], lax.rem(my_pos + 1, N_DEV))

    return pl.pallas_call(
        body,
        out_shape=jax.ShapeDtypeStruct((N_DEV * m_per, n_per), jnp.float32),
        in_specs=[
            pl.BlockSpec(memory_space=pltpu.VMEM),
            pl.BlockSpec(memory_space=pltpu.VMEM),
            pl.BlockSpec(memory_space=pltpu.SMEM),
            pl.BlockSpec(memory_space=pltpu.SMEM),
        ],
        out_specs=pl.BlockSpec(memory_space=pltpu.VMEM),
        scratch_shapes=[
            pltpu.VMEM((N_DEV - 1, m_per, k), jnp.float8_e5m2),
            pltpu.SemaphoreType.DMA((N_DEV - 1,)),
            pltpu.SemaphoreType.DMA((N_DEV - 1,)),
        ],
        compiler_params=pltpu.CompilerParams(collective_id=0),
    )(x8, w8, sx, sw)


# baseline (device time: 219042 ns/iter reference)
import jax
import jax.numpy as jnp
from jax import lax
from jax.experimental import pallas as pl
from jax.experimental.pallas import tpu as pltpu

N_DEV = 4


def kernel(x, w_mat, scale_x, scale_w):
    m_per, k = x.shape
    n_total = w_mat.shape[1]
    n_per = n_total // N_DEV
    my = lax.axis_index("i")

    x8 = x.astype(jnp.float8_e5m2)
    w8 = lax.dynamic_slice(w_mat, (0, my * n_per), (k, n_per)).astype(
        jnp.float8_e5m2
    )
    sx = scale_x.astype(jnp.float32)
    sw = scale_w.astype(jnp.float32)

    def body(x_ref, w_ref, sx_ref, sw_ref, out_ref, comm_ref, send_sems, recv_sems):
        my_pos = lax.axis_index("i")
        left = lax.rem(my_pos + N_DEV - 1, N_DEV)
        right = lax.rem(my_pos + 1, N_DEV)

        barrier_sem = pltpu.get_barrier_semaphore()
        for nbr in (left, right):
            pl.semaphore_signal(
                barrier_sem, inc=1,
                device_id=(nbr,), device_id_type=pl.DeviceIdType.MESH,
            )
        pl.semaphore_wait(barrier_sem, 2)

        scale = sx_ref[0] * sw_ref[0]

        def block(x_chunk, origin):
            acc = jnp.dot(x_chunk, w_ref[...], preferred_element_type=jnp.float32)
            y = acc * scale
            z = y / (1.0 + jnp.exp(-jnp.clip(y, -60.0, 60.0)))
            out_ref[pl.ds(origin * m_per, m_per), :] = z

        hop0 = pltpu.make_async_remote_copy(
            src_ref=x_ref, dst_ref=comm_ref.at[0],
            send_sem=send_sems.at[0], recv_sem=recv_sems.at[0],
            device_id=(right,), device_id_type=pl.DeviceIdType.MESH,
        )
        hop0.start()
        block(x_ref[...], my_pos)
        hop0.wait()

        for h in (1, 2):
            rdma = pltpu.make_async_remote_copy(
                src_ref=comm_ref.at[h - 1], dst_ref=comm_ref.at[h],
                send_sem=send_sems.at[h], recv_sem=recv_sems.at[h],
                device_id=(right,), device_id_type=pl.DeviceIdType.MESH,
            )
            rdma.start()
            block(comm_ref[h - 1], lax.rem(my_pos + N_DEV - h, N_DEV))
            rdma.wait()

        block(comm_ref[2], lax.rem(my_pos + 1, N_DEV))

    return pl.pallas_call(
        body,
        out_shape=jax.ShapeDtypeStruct((N_DEV * m_per, n_per), jnp.float32),
        in_specs=[
            pl.BlockSpec(memory_space=pltpu.VMEM),
            pl.BlockSpec(memory_space=pltpu.VMEM),
            pl.BlockSpec(memory_space=pltpu.SMEM),
            pl.BlockSpec(memory_space=pltpu.SMEM),
        ],
        out_specs=pl.BlockSpec(memory_space=pltpu.VMEM),
        scratch_shapes=[
            pltpu.VMEM((N_DEV - 1, m_per, k), jnp.float8_e5m2),
            pltpu.SemaphoreType.DMA((N_DEV - 1,)),
            pltpu.SemaphoreType.DMA((N_DEV - 1,)),
        ],
        compiler_params=pltpu.CompilerParams(
            collective_id=0, vmem_limit_bytes=128 * 1024 * 1024
        ),
    )(x8, w8, sx, sw)


# device time: 151332 ns/iter; 1.4474x vs baseline; 1.4474x over previous
import jax
import jax.numpy as jnp
from jax import lax
from jax.experimental import pallas as pl
from jax.experimental.pallas import tpu as pltpu

N_DEV = 4


def kernel(x, w_mat, scale_x, scale_w):
    m_per, k = x.shape
    n_total = w_mat.shape[1]
    n_per = n_total // N_DEV
    m_half = m_per // 2
    my = lax.axis_index("i")

    x8 = x.astype(jnp.float8_e5m2)
    w8 = lax.dynamic_slice(w_mat, (0, my * n_per), (k, n_per)).astype(
        jnp.float8_e5m2
    )
    sx = scale_x.astype(jnp.float32)
    sw = scale_w.astype(jnp.float32)

    def body(x_ref, w_ref, sx_ref, sw_ref, out_ref,
             cw_ref, ccw_ref, cw_send, cw_recv, ccw_send, ccw_recv):
        my_pos = lax.axis_index("i")
        left = lax.rem(my_pos + N_DEV - 1, N_DEV)
        right = lax.rem(my_pos + 1, N_DEV)

        barrier_sem = pltpu.get_barrier_semaphore()
        for nbr in (left, right):
            pl.semaphore_signal(
                barrier_sem, inc=1,
                device_id=(nbr,), device_id_type=pl.DeviceIdType.MESH,
            )
        pl.semaphore_wait(barrier_sem, 2)

        scale = sx_ref[0] * sw_ref[0]

        def store(y, origin, row_off, rows):
            y = y * scale
            z = y / (1.0 + jnp.exp(-jnp.clip(y, -60.0, 60.0)))
            out_ref[pl.ds(origin * m_per + row_off, rows), :] = z

        def block_half(x_chunk, origin, half):
            acc = jnp.dot(x_chunk, w_ref[...], preferred_element_type=jnp.float32)
            store(acc, origin, half * m_half, m_half)

        def hop(src_cw, src_ccw, slot):
            cw = pltpu.make_async_remote_copy(
                src_ref=src_cw, dst_ref=cw_ref.at[slot],
                send_sem=cw_send.at[slot], recv_sem=cw_recv.at[slot],
                device_id=(right,), device_id_type=pl.DeviceIdType.MESH,
            )
            ccw = pltpu.make_async_remote_copy(
                src_ref=src_ccw, dst_ref=ccw_ref.at[slot],
                send_sem=ccw_send.at[slot], recv_sem=ccw_recv.at[slot],
                device_id=(left,), device_id_type=pl.DeviceIdType.MESH,
            )
            cw.start()
            ccw.start()
            return cw, ccw

        cw, ccw = hop(x_ref.at[pl.ds(0, m_half)], x_ref.at[pl.ds(m_half, m_half)], 0)
        acc = jnp.dot(x_ref[...], w_ref[...], preferred_element_type=jnp.float32)
        store(acc, my_pos, 0, m_per)

        for h in (1, 2):
            cw.wait()
            ccw.wait()
            cw, ccw = hop(cw_ref.at[h - 1], ccw_ref.at[h - 1], h)
            block_half(cw_ref[h - 1], lax.rem(my_pos + N_DEV - h, N_DEV), 0)
            block_half(ccw_ref[h - 1], lax.rem(my_pos + h, N_DEV), 1)

        cw.wait()
        ccw.wait()
        block_half(cw_ref[2], lax.rem(my_pos + 1, N_DEV), 0)
        block_half(ccw_ref[2], lax.rem(my_pos + 3, N_DEV), 1)

    return pl.pallas_call(
        body,
        out_shape=jax.ShapeDtypeStruct((N_DEV * m_per, n_per), jnp.float32),
        in_specs=[
            pl.BlockSpec(memory_space=pltpu.VMEM),
            pl.BlockSpec(memory_space=pltpu.VMEM),
            pl.BlockSpec(memory_space=pltpu.SMEM),
            pl.BlockSpec(memory_space=pltpu.SMEM),
        ],
        out_specs=pl.BlockSpec(memory_space=pltpu.VMEM),
        scratch_shapes=[
            pltpu.VMEM((N_DEV - 1, m_half, k), jnp.float8_e5m2),
            pltpu.VMEM((N_DEV - 1, m_half, k), jnp.float8_e5m2),
            pltpu.SemaphoreType.DMA((N_DEV - 1,)),
            pltpu.SemaphoreType.DMA((N_DEV - 1,)),
            pltpu.SemaphoreType.DMA((N_DEV - 1,)),
            pltpu.SemaphoreType.DMA((N_DEV - 1,)),
        ],
        compiler_params=pltpu.CompilerParams(
            collective_id=0, vmem_limit_bytes=128 * 1024 * 1024
        ),
    )(x8, w8, sx, sw)


# device time: 123170 ns/iter; 1.7784x vs baseline; 1.2286x over previous
import jax
import jax.numpy as jnp
from jax import lax
from jax.experimental import pallas as pl
from jax.experimental.pallas import tpu as pltpu

N_DEV = 4
K_TILES = 4


def kernel(x, w_mat, scale_x, scale_w):
    m_per, k = x.shape
    n_total = w_mat.shape[1]
    n_per = n_total // N_DEV
    m_half = m_per // 2
    k_tile = k // K_TILES

    x8 = x.astype(jnp.float8_e5m2)
    sx = scale_x.astype(jnp.float32)
    sw = scale_w.astype(jnp.float32)

    def body(x_ref, w_hbm, sx_ref, sw_ref, out_hbm,
             w_stage, w8_ref, out_stage, cw_ref, ccw_ref,
             w_sems, out_sems, cw_send, cw_recv, ccw_send, ccw_recv):
        my_pos = lax.axis_index("i")
        left = lax.rem(my_pos + N_DEV - 1, N_DEV)
        right = lax.rem(my_pos + 1, N_DEV)

        def w_dma(t):
            return pltpu.make_async_copy(
                w_hbm.at[pl.ds(t * k_tile, k_tile), pl.ds(my_pos * n_per, n_per)],
                w_stage.at[t % 2],
                w_sems.at[t % 2],
            )

        w_dmas = [w_dma(t) for t in range(K_TILES)]
        w_dmas[0].start()
        w_dmas[1].start()

        barrier_sem = pltpu.get_barrier_semaphore()
        for nbr in (left, right):
            pl.semaphore_signal(
                barrier_sem, inc=1,
                device_id=(nbr,), device_id_type=pl.DeviceIdType.MESH,
            )
        pl.semaphore_wait(barrier_sem, 2)

        scale = sx_ref[0] * sw_ref[0]

        def hop(src_cw, src_ccw, slot):
            cw = pltpu.make_async_remote_copy(
                src_ref=src_cw, dst_ref=cw_ref.at[slot],
                send_sem=cw_send.at[slot], recv_sem=cw_recv.at[slot],
                device_id=(right,), device_id_type=pl.DeviceIdType.MESH,
            )
            ccw = pltpu.make_async_remote_copy(
                src_ref=src_ccw, dst_ref=ccw_ref.at[slot],
                send_sem=ccw_send.at[slot], recv_sem=ccw_recv.at[slot],
                device_id=(left,), device_id_type=pl.DeviceIdType.MESH,
            )
            cw.start()
            ccw.start()
            return cw, ccw

        cw, ccw = hop(x_ref.at[pl.ds(0, m_half)], x_ref.at[pl.ds(m_half, m_half)], 0)

        for t in range(K_TILES):
            w_dmas[t].wait()
            w8_ref[pl.ds(t * k_tile, k_tile), :] = w_stage[t % 2].astype(
                jnp.float8_e5m2
            )
            if t + 2 < K_TILES:
                w_dmas[t + 2].start()

        out_pending = {}

        def emit(x_chunk_ref, origin, half, slot):
            acc = jnp.dot(
                x_chunk_ref[...], w8_ref[...], preferred_element_type=jnp.float32
            )
            y = acc * scale
            z = y / (1.0 + jnp.exp(-jnp.clip(y, -60.0, 60.0)))
            if slot in out_pending:
                out_pending[slot].wait()
            out_stage[slot] = z
            cp = pltpu.make_async_copy(
                out_stage.at[slot],
                out_hbm.at[pl.ds(origin * m_per + half * m_half, m_half), :],
                out_sems.at[slot],
            )
            cp.start()
            out_pending[slot] = cp

        emit(x_ref.at[pl.ds(0, m_half)], my_pos, 0, 0)

        cw.wait()
        ccw.wait()
        cw, ccw = hop(cw_ref.at[0], ccw_ref.at[0], 1)
        emit(x_ref.at[pl.ds(m_half, m_half)], my_pos, 1, 1)
        emit(cw_ref.at[0], lax.rem(my_pos + 3, N_DEV), 0, 0)
        emit(ccw_ref.at[0], lax.rem(my_pos + 1, N_DEV), 1, 1)

        cw.wait()
        ccw.wait()
        cw, ccw = hop(cw_ref.at[1], ccw_ref.at[1], 2)
        emit(cw_ref.at[1], lax.rem(my_pos + 2, N_DEV), 0, 0)
        emit(ccw_ref.at[1], lax.rem(my_pos + 2, N_DEV), 1, 1)

        cw.wait()
        ccw.wait()
        emit(cw_ref.at[2], lax.rem(my_pos + 1, N_DEV), 0, 0)
        emit(ccw_ref.at[2], lax.rem(my_pos + 3, N_DEV), 1, 1)

        for cp in out_pending.values():
            cp.wait()

    return pl.pallas_call(
        body,
        out_shape=jax.ShapeDtypeStruct((N_DEV * m_per, n_per), jnp.float32),
        in_specs=[
            pl.BlockSpec(memory_space=pltpu.MemorySpace.VMEM),
            pl.BlockSpec(memory_space=pltpu.MemorySpace.HBM),
            pl.BlockSpec(memory_space=pltpu.MemorySpace.SMEM),
            pl.BlockSpec(memory_space=pltpu.MemorySpace.SMEM),
        ],
        out_specs=pl.BlockSpec(memory_space=pltpu.MemorySpace.HBM),
        scratch_shapes=[
            pltpu.VMEM((2, k_tile, n_per), jnp.float32),
            pltpu.VMEM((k, n_per), jnp.float8_e5m2),
            pltpu.VMEM((2, m_half, n_per), jnp.float32),
            pltpu.VMEM((N_DEV - 1, m_half, k), jnp.float8_e5m2),
            pltpu.VMEM((N_DEV - 1, m_half, k), jnp.float8_e5m2),
            pltpu.SemaphoreType.DMA((2,)),
            pltpu.SemaphoreType.DMA((2,)),
            pltpu.SemaphoreType.DMA((N_DEV - 1,)),
            pltpu.SemaphoreType.DMA((N_DEV - 1,)),
            pltpu.SemaphoreType.DMA((N_DEV - 1,)),
            pltpu.SemaphoreType.DMA((N_DEV - 1,)),
        ],
        compiler_params=pltpu.CompilerParams(
            collective_id=0, vmem_limit_bytes=128 * 1024 * 1024
        ),
    )(x8, w_mat, sx, sw)


# device time: 123059 ns/iter; 1.7800x vs baseline; 1.0009x over previous
import jax
import jax.numpy as jnp
from jax import lax
from jax.experimental import pallas as pl
from jax.experimental.pallas import tpu as pltpu

N_DEV = 4
K_TILES = 4


def kernel(x, w_mat, scale_x, scale_w):
    m_per, k = x.shape
    n_total = w_mat.shape[1]
    n_per = n_total // N_DEV
    m_half = m_per // 2
    k_tile = k // K_TILES

    x8 = x.astype(jnp.float8_e5m2)
    sx = scale_x.astype(jnp.float32)
    sw = scale_w.astype(jnp.float32)

    def body(x_ref, w_hbm, sx_ref, sw_ref, out_hbm,
             w_stage, w8_ref, out_stage, cw_ref, ccw_ref,
             w_sems, out_sems, cw_send, cw_recv, ccw_send, ccw_recv):
        my_pos = lax.axis_index("i")
        left = lax.rem(my_pos + N_DEV - 1, N_DEV)
        right = lax.rem(my_pos + 1, N_DEV)

        def w_dma(t):
            return pltpu.make_async_copy(
                w_hbm.at[pl.ds(t * k_tile, k_tile), pl.ds(my_pos * n_per, n_per)],
                w_stage.at[t % 2],
                w_sems.at[t % 2],
            )

        w_dmas = [w_dma(t) for t in range(K_TILES)]
        w_dmas[0].start()
        w_dmas[1].start()

        barrier_sem = pltpu.get_barrier_semaphore()
        for nbr in (left, right):
            pl.semaphore_signal(
                barrier_sem, inc=1,
                device_id=(nbr,), device_id_type=pl.DeviceIdType.MESH,
            )
        pl.semaphore_wait(barrier_sem, 2)

        scale = sx_ref[0] * sw_ref[0]

        def hop(src_cw, src_ccw, slot):
            cw = pltpu.make_async_remote_copy(
                src_ref=src_cw, dst_ref=cw_ref.at[slot],
                send_sem=cw_send.at[slot], recv_sem=cw_recv.at[slot],
                device_id=(right,), device_id_type=pl.DeviceIdType.MESH,
            )
            ccw = pltpu.make_async_remote_copy(
                src_ref=src_ccw, dst_ref=ccw_ref.at[slot],
                send_sem=ccw_send.at[slot], recv_sem=ccw_recv.at[slot],
                device_id=(left,), device_id_type=pl.DeviceIdType.MESH,
            )
            cw.start()
            ccw.start()
            return cw, ccw

        cw, ccw = hop(x_ref.at[pl.ds(0, m_half)], x_ref.at[pl.ds(m_half, m_half)], 0)

        out_pending = {}

        def flush(z, origin, half, slot):
            if slot in out_pending:
                out_pending[slot].wait()
            out_stage[slot] = z
            cp = pltpu.make_async_copy(
                out_stage.at[slot],
                out_hbm.at[pl.ds(origin * m_per + half * m_half, m_half), :],
                out_sems.at[slot],
            )
            cp.start()
            out_pending[slot] = cp

        def silu(acc):
            y = acc * scale
            return y / (1.0 + jnp.exp(-jnp.clip(y, -60.0, 60.0)))

        def emit(x_chunk_ref, origin, half, slot):
            acc = jnp.dot(
                x_chunk_ref[...], w8_ref[...], preferred_element_type=jnp.float32
            )
            flush(silu(acc), origin, half, slot)

        for t in range(K_TILES):
            w_dmas[t].wait()
            wt = w_stage[t % 2].astype(jnp.float8_e5m2)
            w8_ref[pl.ds(t * k_tile, k_tile), :] = wt
            if t + 2 < K_TILES:
                w_dmas[t + 2].start()
            ks = pl.ds(t * k_tile, k_tile)
            p_lo = jnp.dot(
                x_ref[pl.ds(0, m_half), ks], wt, preferred_element_type=jnp.float32
            )
            p_hi = jnp.dot(
                x_ref[pl.ds(m_half, m_half), ks], wt,
                preferred_element_type=jnp.float32,
            )
            if t == 0:
                out_stage[0] = p_lo
                out_stage[1] = p_hi
            else:
                out_stage[0] = out_stage[0] + p_lo
                out_stage[1] = out_stage[1] + p_hi

        cw.wait()
        ccw.wait()
        cw, ccw = hop(cw_ref.at[0], ccw_ref.at[0], 1)
        flush(silu(out_stage[0]), my_pos, 0, 0)
        flush(silu(out_stage[1]), my_pos, 1, 1)
        emit(cw_ref.at[0], lax.rem(my_pos + 3, N_DEV), 0, 0)
        emit(ccw_ref.at[0], lax.rem(my_pos + 1, N_DEV), 1, 1)

        cw.wait()
        ccw.wait()
        cw, ccw = hop(cw_ref.at[1], ccw_ref.at[1], 2)
        emit(cw_ref.at[1], lax.rem(my_pos + 2, N_DEV), 0, 0)
        emit(ccw_ref.at[1], lax.rem(my_pos + 2, N_DEV), 1, 1)

        cw.wait()
        ccw.wait()
        emit(cw_ref.at[2], lax.rem(my_pos + 1, N_DEV), 0, 0)
        emit(ccw_ref.at[2], lax.rem(my_pos + 3, N_DEV), 1, 1)

        for cp in out_pending.values():
            cp.wait()

    return pl.pallas_call(
        body,
        out_shape=jax.ShapeDtypeStruct((N_DEV * m_per, n_per), jnp.float32),
        in_specs=[
            pl.BlockSpec(memory_space=pltpu.MemorySpace.VMEM),
            pl.BlockSpec(memory_space=pltpu.MemorySpace.HBM),
            pl.BlockSpec(memory_space=pltpu.MemorySpace.SMEM),
            pl.BlockSpec(memory_space=pltpu.MemorySpace.SMEM),
        ],
        out_specs=pl.BlockSpec(memory_space=pltpu.MemorySpace.HBM),
        scratch_shapes=[
            pltpu.VMEM((2, k_tile, n_per), jnp.float32),
            pltpu.VMEM((k, n_per), jnp.float8_e5m2),
            pltpu.VMEM((2, m_half, n_per), jnp.float32),
            pltpu.VMEM((N_DEV - 1, m_half, k), jnp.float8_e5m2),
            pltpu.VMEM((N_DEV - 1, m_half, k), jnp.float8_e5m2),
            pltpu.SemaphoreType.DMA((2,)),
            pltpu.SemaphoreType.DMA((2,)),
            pltpu.SemaphoreType.DMA((N_DEV - 1,)),
            pltpu.SemaphoreType.DMA((N_DEV - 1,)),
            pltpu.SemaphoreType.DMA((N_DEV - 1,)),
            pltpu.SemaphoreType.DMA((N_DEV - 1,)),
        ],
        compiler_params=pltpu.CompilerParams(
            collective_id=0, vmem_limit_bytes=128 * 1024 * 1024
        ),
    )(x8, w_mat, sx, sw)


# device time: 108245 ns/iter; 2.0236x vs baseline; 1.1369x over previous
import jax
import jax.numpy as jnp
from jax import lax
from jax.experimental import pallas as pl
from jax.experimental.pallas import tpu as pltpu

N_DEV = 4
SUBS_PER_HALF = 2
N_SLOTS = (N_DEV - 1) * SUBS_PER_HALF
W_TILES = 8


def kernel(x, w_mat, scale_x, scale_w):
    m_per, k = x.shape
    n_total = w_mat.shape[1]
    n_per = n_total // N_DEV
    m_half = m_per // 2
    m_sub = m_half // SUBS_PER_HALF
    w_kt = k // W_TILES

    sx = scale_x.astype(jnp.float32)
    sw = scale_w.astype(jnp.float32)

    def body(x_hbm, w_hbm, sx_ref, sw_ref, out_hbm,
             stage, x8_ref, w8_ref, acc_ref, out_stage,
             stage_sems, own_sems, out_sems,
             cw_send, cw_recv, ccw_send, ccw_recv, cw_ref, ccw_ref):
        my_pos = lax.axis_index("i")
        left = lax.rem(my_pos + N_DEV - 1, N_DEV)
        right = lax.rem(my_pos + 1, N_DEV)

        xd = [
            pltpu.make_async_copy(
                x_hbm.at[pl.ds(s * m_half, m_half)], stage.at[s], stage_sems.at[s]
            )
            for s in (0, 1)
        ]
        xd[0].start()
        xd[1].start()

        barrier_sem = pltpu.get_barrier_semaphore()
        for nbr in (left, right):
            pl.semaphore_signal(
                barrier_sem, inc=1,
                device_id=(nbr,), device_id_type=pl.DeviceIdType.MESH,
            )
        pl.semaphore_wait(barrier_sem, 2)

        scale = sx_ref[0] * sw_ref[0]

        def remote(src, dst, ssem, rsem, dev):
            r = pltpu.make_async_remote_copy(
                src_ref=src, dst_ref=dst, send_sem=ssem, recv_sem=rsem,
                device_id=(dev,), device_id_type=pl.DeviceIdType.MESH,
            )
            r.start()
            return r

        def w_dma(t):
            return pltpu.make_async_copy(
                w_hbm.at[pl.ds(t * w_kt, w_kt), pl.ds(my_pos * n_per, n_per)],
                stage.at[t % 2, :, pl.ds(0, n_per)],
                stage_sems.at[t % 2],
            )

        cur_cw, cur_ccw, wd = {}, {}, {}
        xd[0].wait()
        x8_ref[pl.ds(0, m_half), :] = stage[0].astype(jnp.float8_e5m2)
        for s in range(SUBS_PER_HALF):
            cur_cw[s] = remote(
                x8_ref.at[pl.ds(s * m_sub, m_sub)], cw_ref.at[s],
                cw_send.at[s], cw_recv.at[s], right,
            )
        wd[0] = w_dma(0)
        wd[0].start()
        xd[1].wait()
        x8_ref[pl.ds(m_half, m_half), :] = stage[1].astype(jnp.float8_e5m2)
        for s in range(SUBS_PER_HALF):
            cur_ccw[s] = remote(
                x8_ref.at[pl.ds(m_half + s * m_sub, m_sub)], ccw_ref.at[s],
                ccw_send.at[s], ccw_recv.at[s], left,
            )
        wd[1] = w_dma(1)
        wd[1].start()

        for t in range(W_TILES):
            wd[t].wait()
            wt = stage[t % 2, :, pl.ds(0, n_per)].astype(jnp.float8_e5m2)
            w8_ref[pl.ds(t * w_kt, w_kt), :] = wt
            if t + 2 < W_TILES:
                wd[t + 2] = w_dma(t + 2)
                wd[t + 2].start()
            ks = pl.ds(t * w_kt, w_kt)
            p_lo = jnp.dot(
                x8_ref[pl.ds(0, m_half), ks], wt, preferred_element_type=jnp.float32
            )
            p_hi = jnp.dot(
                x8_ref[pl.ds(m_half, m_half), ks], wt,
                preferred_element_type=jnp.float32,
            )
            if t == 0:
                acc_ref[0] = p_lo
                acc_ref[1] = p_hi
            else:
                acc_ref[0] = acc_ref[0] + p_lo
                acc_ref[1] = acc_ref[1] + p_hi

        def silu(y):
            return y / (1.0 + jnp.exp(-jnp.clip(y, -60.0, 60.0)))

        out_pending = {}
        own_cps = []

        def emit_sub(src_ref, origin, half, sub, slot):
            acc = jnp.dot(
                src_ref[...], w8_ref[...], preferred_element_type=jnp.float32
            )
            z = silu(acc * scale)
            if slot in out_pending:
                out_pending[slot].wait()
            out_stage[slot] = z
            cp = pltpu.make_async_copy(
                out_stage.at[slot],
                out_hbm.at[
                    pl.ds(origin * m_per + half * m_half + sub * m_sub, m_sub), :
                ],
                out_sems.at[slot],
            )
            cp.start()
            out_pending[slot] = cp

        for j in range(N_SLOTS):
            cur_cw[j].wait()
            if j + SUBS_PER_HALF < N_SLOTS:
                cur_cw[j + 2] = remote(
                    cw_ref.at[j], cw_ref.at[j + 2],
                    cw_send.at[j + 2], cw_recv.at[j + 2], right,
                )
            cur_ccw[j].wait()
            if j + SUBS_PER_HALF < N_SLOTS:
                cur_ccw[j + 2] = remote(
                    ccw_ref.at[j], ccw_ref.at[j + 2],
                    ccw_send.at[j + 2], ccw_recv.at[j + 2], left,
                )
            if j == 0:
                for h in (0, 1):
                    acc_ref[h] = silu(acc_ref[h] * scale)
                    cp = pltpu.make_async_copy(
                        acc_ref.at[h],
                        out_hbm.at[pl.ds(my_pos * m_per + h * m_half, m_half), :],
                        own_sems.at[h],
                    )
                    cp.start()
                    own_cps.append(cp)
            hopn = j // 2 + 1
            emit_sub(
                cw_ref.at[j], lax.rem(my_pos + N_DEV - hopn, N_DEV),
                0, j % 2, (2 * j) % 3,
            )
            emit_sub(
                ccw_ref.at[j], lax.rem(my_pos + hopn, N_DEV),
                1, j % 2, (2 * j + 1) % 3,
            )

        for cp in own_cps:
            cp.wait()
        for cp in out_pending.values():
            cp.wait()

    return pl.pallas_call(
        body,
        out_shape=jax.ShapeDtypeStruct((N_DEV * m_per, n_per), jnp.float32),
        in_specs=[
            pl.BlockSpec(memory_space=pltpu.MemorySpace.HBM),
            pl.BlockSpec(memory_space=pltpu.MemorySpace.HBM),
            pl.BlockSpec(memory_space=pltpu.MemorySpace.SMEM),
            pl.BlockSpec(memory_space=pltpu.MemorySpace.SMEM),
        ],
        out_specs=pl.BlockSpec(memory_space=pltpu.MemorySpace.HBM),
        scratch_shapes=[
            pltpu.VMEM((2, m_half, k), jnp.float32),
            pltpu.VMEM((m_per, k), jnp.float8_e5m2),
            pltpu.VMEM((k, n_per), jnp.float8_e5m2),
            pltpu.VMEM((2, m_half, n_per), jnp.float32),
            pltpu.VMEM((3, m_sub, n_per), jnp.float32),
            pltpu.SemaphoreType.DMA((2,)),
            pltpu.SemaphoreType.DMA((2,)),
            pltpu.SemaphoreType.DMA((3,)),
            pltpu.SemaphoreType.DMA((N_SLOTS,)),
            pltpu.SemaphoreType.DMA((N_SLOTS,)),
            pltpu.SemaphoreType.DMA((N_SLOTS,)),
            pltpu.SemaphoreType.DMA((N_SLOTS,)),
            pltpu.VMEM((N_SLOTS, m_sub, k), jnp.float8_e5m2),
            pltpu.VMEM((N_SLOTS, m_sub, k), jnp.float8_e5m2),
        ],
        compiler_params=pltpu.CompilerParams(
            collective_id=0, vmem_limit_bytes=128 * 1024 * 1024
        ),
    )(x, w_mat, sx, sw)


# device time: 106070 ns/iter; 2.0651x vs baseline; 1.0205x over previous
import jax
import jax.numpy as jnp
from jax import lax
from jax.experimental import pallas as pl
from jax.experimental.pallas import tpu as pltpu

N_DEV = 4
SUBS_PER_HALF = 2
N_SLOTS = (N_DEV - 1) * SUBS_PER_HALF
W_TILES = 8


def kernel(x, w_mat, scale_x, scale_w):
    m_per, k = x.shape
    n_total = w_mat.shape[1]
    n_per = n_total // N_DEV
    m_half = m_per // 2
    m_sub = m_half // SUBS_PER_HALF
    w_kt = k // W_TILES

    sx = scale_x.astype(jnp.float32)
    sw = scale_w.astype(jnp.float32)

    def body(x_hbm, w_hbm, sx_ref, sw_ref, out_hbm,
             stage, x8_ref, w8_ref, acc_ref, out_stage,
             stage_sems, own_sems, out_sems,
             cw_send, cw_recv, ccw_send, ccw_recv, cw_ref, ccw_ref):
        my_pos = lax.axis_index("i")
        left = lax.rem(my_pos + N_DEV - 1, N_DEV)
        right = lax.rem(my_pos + 1, N_DEV)

        xd = [
            pltpu.make_async_copy(
                x_hbm.at[pl.ds(q * m_sub, m_sub)],
                stage.at[q // 2, pl.ds((q % 2) * m_sub, m_sub), :],
                stage_sems.at[q],
            )
            for q in range(4)
        ]
        for d in xd:
            d.start()

        barrier_sem = pltpu.get_barrier_semaphore()
        for nbr in (left, right):
            pl.semaphore_signal(
                barrier_sem, inc=1,
                device_id=(nbr,), device_id_type=pl.DeviceIdType.MESH,
            )
        pl.semaphore_wait(barrier_sem, 2)

        scale = sx_ref[0] * sw_ref[0]

        def remote(src, dst, ssem, rsem, dev):
            r = pltpu.make_async_remote_copy(
                src_ref=src, dst_ref=dst, send_sem=ssem, recv_sem=rsem,
                device_id=(dev,), device_id_type=pl.DeviceIdType.MESH,
            )
            r.start()
            return r

        def w_dma(t):
            return pltpu.make_async_copy(
                w_hbm.at[pl.ds(t * w_kt, w_kt), pl.ds(my_pos * n_per, n_per)],
                stage.at[t % 2, :, pl.ds(0, n_per)],
                stage_sems.at[t % 2],
            )

        cur_cw, cur_ccw, wd = {}, {}, {}
        for q in (0, 2, 1, 3):
            xd[q].wait()
            rows = pl.ds(q * m_sub, m_sub)
            x8_ref[rows, :] = stage[
                q // 2, pl.ds((q % 2) * m_sub, m_sub), :
            ].astype(jnp.float8_e5m2)
            if q < 2:
                cur_cw[q] = remote(
                    x8_ref.at[rows], cw_ref.at[q],
                    cw_send.at[q], cw_recv.at[q], right,
                )
            else:
                cur_ccw[q - 2] = remote(
                    x8_ref.at[rows], ccw_ref.at[q - 2],
                    ccw_send.at[q - 2], ccw_recv.at[q - 2], left,
                )
        wd[0] = w_dma(0)
        wd[0].start()
        wd[1] = w_dma(1)
        wd[1].start()

        for t in range(W_TILES):
            wd[t].wait()
            wt = stage[t % 2, :, pl.ds(0, n_per)].astype(jnp.float8_e5m2)
            w8_ref[pl.ds(t * w_kt, w_kt), :] = wt
            if t + 2 < W_TILES:
                wd[t + 2] = w_dma(t + 2)
                wd[t + 2].start()
            ks = pl.ds(t * w_kt, w_kt)
            p_lo = jnp.dot(
                x8_ref[pl.ds(0, m_half), ks], wt, preferred_element_type=jnp.float32
            )
            p_hi = jnp.dot(
                x8_ref[pl.ds(m_half, m_half), ks], wt,
                preferred_element_type=jnp.float32,
            )
            if t == 0:
                acc_ref[0] = p_lo
                acc_ref[1] = p_hi
            else:
                acc_ref[0] = acc_ref[0] + p_lo
                acc_ref[1] = acc_ref[1] + p_hi

        def silu(y):
            return y / (1.0 + jnp.exp(-jnp.clip(y, -60.0, 60.0)))

        out_pending = {}
        own_cps = []

        def emit_sub(src_ref, origin, half, sub, slot):
            acc = jnp.dot(
                src_ref[...], w8_ref[...], preferred_element_type=jnp.float32
            )
            z = silu(acc * scale)
            if slot in out_pending:
                out_pending[slot].wait()
            out_stage[slot] = z
            cp = pltpu.make_async_copy(
                out_stage.at[slot],
                out_hbm.at[
                    pl.ds(origin * m_per + half * m_half + sub * m_sub, m_sub), :
                ],
                out_sems.at[slot],
            )
            cp.start()
            out_pending[slot] = cp

        for j in range(N_SLOTS):
            cur_cw[j].wait()
            if j + SUBS_PER_HALF < N_SLOTS:
                cur_cw[j + 2] = remote(
                    cw_ref.at[j], cw_ref.at[j + 2],
                    cw_send.at[j + 2], cw_recv.at[j + 2], right,
                )
            cur_ccw[j].wait()
            if j + SUBS_PER_HALF < N_SLOTS:
                cur_ccw[j + 2] = remote(
                    ccw_ref.at[j], ccw_ref.at[j + 2],
                    ccw_send.at[j + 2], ccw_recv.at[j + 2], left,
                )
            if j == 0:
                for h in (0, 1):
                    acc_ref[h] = silu(acc_ref[h] * scale)
                    cp = pltpu.make_async_copy(
                        acc_ref.at[h],
                        out_hbm.at[pl.ds(my_pos * m_per + h * m_half, m_half), :],
                        own_sems.at[h],
                    )
                    cp.start()
                    own_cps.append(cp)
            hopn = j // 2 + 1
            emit_sub(
                cw_ref.at[j], lax.rem(my_pos + N_DEV - hopn, N_DEV),
                0, j % 2, (2 * j) % 3,
            )
            emit_sub(
                ccw_ref.at[j], lax.rem(my_pos + hopn, N_DEV),
                1, j % 2, (2 * j + 1) % 3,
            )

        for cp in own_cps:
            cp.wait()
        for cp in out_pending.values():
            cp.wait()

    return pl.pallas_call(
        body,
        out_shape=jax.ShapeDtypeStruct((N_DEV * m_per, n_per), jnp.float32),
        in_specs=[
            pl.BlockSpec(memory_space=pltpu.MemorySpace.HBM),
            pl.BlockSpec(memory_space=pltpu.MemorySpace.HBM),
            pl.BlockSpec(memory_space=pltpu.MemorySpace.SMEM),
            pl.BlockSpec(memory_space=pltpu.MemorySpace.SMEM),
        ],
        out_specs=pl.BlockSpec(memory_space=pltpu.MemorySpace.HBM),
        scratch_shapes=[
            pltpu.VMEM((2, m_half, k), jnp.float32),
            pltpu.VMEM((m_per, k), jnp.float8_e5m2),
            pltpu.VMEM((k, n_per), jnp.float8_e5m2),
            pltpu.VMEM((2, m_half, n_per), jnp.float32),
            pltpu.VMEM((3, m_sub, n_per), jnp.float32),
            pltpu.SemaphoreType.DMA((4,)),
            pltpu.SemaphoreType.DMA((2,)),
            pltpu.SemaphoreType.DMA((3,)),
            pltpu.SemaphoreType.DMA((N_SLOTS,)),
            pltpu.SemaphoreType.DMA((N_SLOTS,)),
            pltpu.SemaphoreType.DMA((N_SLOTS,)),
            pltpu.SemaphoreType.DMA((N_SLOTS,)),
            pltpu.VMEM((N_SLOTS, m_sub, k), jnp.float8_e5m2),
            pltpu.VMEM((N_SLOTS, m_sub, k), jnp.float8_e5m2),
        ],
        compiler_params=pltpu.CompilerParams(
            collective_id=0, vmem_limit_bytes=128 * 1024 * 1024
        ),
    )(x, w_mat, sx, sw)
